# baseline (speedup 1.0000x reference)
import functools

import jax
import jax.numpy as jnp
from jax import lax
from jax.experimental import pallas as pl
from jax.experimental.pallas import tpu as pltpu
from jax.experimental.pallas import tpu_sc as plsc

B, T, D = 16, 2048, 256
VOCAB = 100000

_NC = 2
_NS = 16
_NW = _NC * _NS
_N_IDX = B * T
_CHUNK = _N_IDX // _NW
_SUB = 8
_SUBW = _CHUNK // _SUB


def _sc_gather(weights, idx3):
    mesh = plsc.VectorSubcoreMesh(core_axis_name="c", subcore_axis_name="s")
    nw, sub, subw = idx3.shape

    @functools.partial(
        pl.kernel,
        mesh=mesh,
        out_type=jax.ShapeDtypeStruct((nw, sub, subw), jnp.float32),
        scratch_types=[
            pltpu.VMEM((sub, subw), jnp.int32),
            pltpu.VMEM((sub, subw), jnp.float32),
            pltpu.SemaphoreType.DMA,
        ],
        compiler_params=pltpu.CompilerParams(needs_layout_passes=False),
    )
    def gather_kernel(w_hbm, idx_hbm, out_hbm, idx_v, rows_v, sem):
        wid = lax.axis_index("s") * _NC + lax.axis_index("c")
        pltpu.sync_copy(idx_hbm.at[wid], idx_v)
        copies = [
            pltpu.async_copy(w_hbm.at[idx_v.at[j]], rows_v.at[j], sem)
            for j in range(sub)
        ]
        for c in copies:
            c.wait()
        pltpu.sync_copy(rows_v, out_hbm.at[wid])

    return gather_kernel(weights, idx3)


_NROW = 4
_GB = B // _NROW
_HT = T // 2


def _one_row(length, vs_lo, vs_hi, w_row_raw, y_ref, yh_ref):
    pos = lax.broadcasted_iota(jnp.int32, (1, T), 1)
    maskf = (pos < length).astype(jnp.float32)
    w_row = w_row_raw * maskf
    a = jnp.concatenate([maskf, w_row], axis=0)
    acc = (
        jnp.dot(a[:, :_HT], vs_lo, preferred_element_type=jnp.float32)
        + jnp.dot(a[:, _HT:], vs_hi, preferred_element_type=jnp.float32)
    )
    s = acc[0:1, :]
    denom = jnp.sqrt(jnp.sum(jnp.abs(s)))
    y_ref[0, :, :] = s / denom
    yh_ref[0, :, :] = acc[1:2, :]


def _tc_body(len_ref, *refs):
    vs_refs = refs[:2 * _NROW]
    w_refs = refs[2 * _NROW:3 * _NROW]
    y_refs = refs[3 * _NROW:4 * _NROW]
    yh_refs = refs[4 * _NROW:]
    b = pl.program_id(0)
    for k in range(_NROW):
        _one_row(len_ref[b + k * _GB],
                 vs_refs[2 * k][0, 0], vs_refs[2 * k + 1][0, 0],
                 w_refs[k][0], y_refs[k], yh_refs[k])


def kernel(vector_sequence, sentence_length, word_sequence, weights):
    idx3 = word_sequence.astype(jnp.int32).reshape(_NW, _SUB, _SUBW)
    w_tok = _sc_gather(weights, idx3)
    w3 = w_tok.reshape(B, 1, T)
    lens = sentence_length.astype(jnp.int32)
    vs4 = vector_sequence.reshape(B, 2, _HT, D)

    def _off(k, half):
        return lambda b: (b + k * _GB, half, 0, 0)

    def _woff(k):
        return lambda b: (b + k * _GB, 0, 0)

    vs_specs = []
    for k in range(_NROW):
        vs_specs.append(pl.BlockSpec((1, 1, _HT, D), _off(k, 0)))
        vs_specs.append(pl.BlockSpec((1, 1, _HT, D), _off(k, 1)))
    w_specs = [pl.BlockSpec((1, 1, T), _woff(k)) for k in range(_NROW)]
    out_spec = pl.BlockSpec((1, 1, D), lambda b: (b, 0, 0))
    out_ty = jax.ShapeDtypeStruct((_GB, 1, D), jnp.float32)
    outs = pl.pallas_call(
        _tc_body,
        grid=(_GB,),
        in_specs=[
            pl.BlockSpec(memory_space=pltpu.SMEM),
            *vs_specs,
            *w_specs,
        ],
        out_specs=[out_spec] * (2 * _NROW),
        out_shape=[out_ty] * (2 * _NROW),
    )(lens, *([vs4] * (2 * _NROW)), *([w3] * _NROW))
    y = jnp.concatenate(outs[:_NROW], axis=0).reshape(B, D)
    y_hat = jnp.concatenate(outs[_NROW:], axis=0).reshape(B, D)
    return y, y_hat


# speedup vs baseline: 1.0043x; 1.0043x over previous
import functools

import jax
import jax.numpy as jnp
from jax import lax
from jax.experimental import pallas as pl
from jax.experimental.pallas import tpu as pltpu
from jax.experimental.pallas import tpu_sc as plsc

B, T, D = 16, 2048, 256
VOCAB = 100000

_NC = 2
_NS = 16
_NW = _NC * _NS
_N_IDX = B * T
_CHUNK = _N_IDX // _NW
_SUB = 8
_SUBW = _CHUNK // _SUB


def _sc_gather(weights, idx3):
    mesh = plsc.VectorSubcoreMesh(core_axis_name="c", subcore_axis_name="s")
    nw, sub, subw = idx3.shape

    @functools.partial(
        pl.kernel,
        mesh=mesh,
        out_type=jax.ShapeDtypeStruct((nw, sub, subw), jnp.float32),
        scratch_types=[
            pltpu.VMEM((sub, subw), jnp.int32),
            pltpu.VMEM((sub, subw), jnp.float32),
            pltpu.SemaphoreType.DMA,
            pltpu.SemaphoreType.DMA,
            pltpu.SemaphoreType.DMA,
        ],
        compiler_params=pltpu.CompilerParams(needs_layout_passes=False),
    )
    def gather_kernel(w_hbm, idx_hbm, out_hbm, idx_v, rows_v,
                      sem_i, sem_g, sem_o):
        wid = lax.axis_index("s") * _NC + lax.axis_index("c")
        icps = [
            pltpu.async_copy(idx_hbm.at[wid, j], idx_v.at[j], sem_i)
            for j in range(sub)
        ]
        gcps = []
        for j in range(sub):
            icps[j].wait()
            gcps.append(
                pltpu.async_copy(w_hbm.at[idx_v.at[j]], rows_v.at[j], sem_g))
        ocps = []
        for j in range(sub):
            gcps[j].wait()
            ocps.append(
                pltpu.async_copy(rows_v.at[j], out_hbm.at[wid, j], sem_o))
        for o in ocps:
            o.wait()

    return gather_kernel(weights, idx3)


_NROW = 4
_GB = B // _NROW


def _one_row(length, vs, w_row_raw, y_ref, yh_ref):
    pos = lax.broadcasted_iota(jnp.int32, (1, T), 1)
    maskf = (pos < length).astype(jnp.float32)
    w_row = w_row_raw * maskf
    a = jnp.concatenate([maskf, w_row], axis=0)
    acc = jnp.dot(a, vs, preferred_element_type=jnp.float32)
    s = acc[0:1, :]
    denom = jnp.sqrt(jnp.sum(jnp.abs(s)))
    y_ref[0, :, :] = s / denom
    yh_ref[0, :, :] = acc[1:2, :]


def _tc_body(len_ref, *refs):
    vs_refs = refs[:_NROW]
    w_refs = refs[_NROW:2 * _NROW]
    y_refs = refs[2 * _NROW:3 * _NROW]
    yh_refs = refs[3 * _NROW:]
    b = pl.program_id(0)
    for k in range(_NROW):
        _one_row(len_ref[b + k * _GB], vs_refs[k][0], w_refs[k][0],
                 y_refs[k], yh_refs[k])


def kernel(vector_sequence, sentence_length, word_sequence, weights):
    idx3 = word_sequence.astype(jnp.int32).reshape(_NW, _SUB, _SUBW)
    w_tok = _sc_gather(weights, idx3)
    w3 = w_tok.reshape(B, 1, T)
    lens = sentence_length.astype(jnp.int32)

    def _off(k):
        return lambda b: (b + k * _GB, 0, 0)

    vs_specs = [pl.BlockSpec((1, T, D), _off(k)) for k in range(_NROW)]
    w_specs = [pl.BlockSpec((1, 1, T), _off(k)) for k in range(_NROW)]
    out_spec = pl.BlockSpec((1, 1, D), lambda b: (b, 0, 0))
    out_ty = jax.ShapeDtypeStruct((_GB, 1, D), jnp.float32)
    outs = pl.pallas_call(
        _tc_body,
        grid=(_GB,),
        in_specs=[
            pl.BlockSpec(memory_space=pltpu.SMEM),
            *vs_specs,
            *w_specs,
        ],
        out_specs=[out_spec] * (2 * _NROW),
        out_shape=[out_ty] * (2 * _NROW),
    )(lens, *([vector_sequence] * _NROW), *([w3] * _NROW))
    y = jnp.concatenate(outs[:_NROW], axis=0).reshape(B, D)
    y_hat = jnp.concatenate(outs[_NROW:], axis=0).reshape(B, D)
    return y, y_hat
